# baseline (speedup 1.0000x reference)
import functools
import math

import jax
import jax.numpy as jnp
from jax import lax
from jax.experimental import pallas as pl
from jax.experimental.pallas import tpu as pltpu
from jax.experimental.pallas import tpu_sc as plsc

_NUM_BUCKETS = 32
_NUM_HEADS = 16
_MAX_DIST = 128
_SEQ = 2048
_NDIAG = 2 * _SEQ - 1
_BROWS = 33
_BTAB = _BROWS * 128
_VROW = 4112
_HALF_ROWS = _SEQ // 2


def _bucket_tc_body(delta_ref, out_ref):
    flat = lax.broadcasted_iota(jnp.int32, (_BTAB,), 0)
    rel = flat - (_SEQ - 1) + delta_ref[0]
    half = _NUM_BUCKETS // 2
    buckets = (rel > 0).astype(jnp.int32) * half
    arel = jnp.abs(rel)
    max_exact = half // 2
    large = max_exact + (
        jnp.log(arel.astype(jnp.float32) / max_exact)
        / math.log(_MAX_DIST / max_exact)
        * (half - max_exact)
    ).astype(jnp.int32)
    large = jnp.minimum(large, half - 1)
    out_ref[...] = buckets + jnp.where(arel < max_exact, arel, large)


_bucket_tc = pl.pallas_call(
    _bucket_tc_body,
    out_shape=jax.ShapeDtypeStruct((_BTAB,), jnp.int32),
    in_specs=[pl.BlockSpec(memory_space=pltpu.SMEM)],
    out_specs=pl.BlockSpec(memory_space=pltpu.VMEM),
)


def _sc_materialize_body(
    btab_hbm, emb_hbm, out_hbm, btab_v, emb_v, vtab0,
    stage0, stage1, stage2, stage3, stage4, stage5,
    sem0, sem1, sem2, sem3, sem4, sem5
):
    stages = (stage0, stage1, stage2, stage3, stage4, stage5)
    sems = (sem0, sem1, sem2, sem3, sem4, sem5)
    cid = lax.axis_index("c")
    sid = lax.axis_index("s")
    wid = sid * 2 + cid
    head = wid // 2
    half = wid - head * 2
    c0 = pltpu.async_copy(btab_hbm, btab_v, sem0)
    c1 = pltpu.async_copy(emb_hbm, emb_v, sem1)
    c0.wait()
    c1.wait()
    iota = lax.iota(jnp.int32, 16)
    hvec = jnp.zeros((16,), jnp.int32) + head
    row0 = half * _HALF_ROWS

    @plsc.parallel_loop(0, _VROW // 16, unroll=4)
    def _build(k):
        bidx = btab_v[pl.ds(k * 16, 16)]
        vtab0[pl.ds(k * 16, 16)] = plsc.load_gather(emb_v, [bidx, hvec])

    nbuf = len(stages)
    rows_blk = 8

    def one_block(k, buf, wait):
        i0 = row0 + k * rows_blk
        stage = stages[buf]

        @pl.when(wait)
        def _wait_prev():
            pltpu.make_async_copy(
                stage, out_hbm.at[pl.ds(0, rows_blk)], sems[buf]
            ).wait()

        start0 = _SEQ - 1 - i0

        @plsc.parallel_loop(0, _SEQ // 16, unroll=8)
        def _fill(c):
            c16 = c * 16
            idx0 = start0 + c16 + iota
            for r in range(rows_blk):
                stage[r, pl.ds(c16, 16)] = plsc.load_gather(vtab0, [idx0 - r])

        pltpu.async_copy(
            stage, out_hbm.at[pl.ds(head * _SEQ + i0, rows_blk)], sems[buf]
        )

    nblocks = _HALF_ROWS // rows_blk

    def block(bk, carry):
        for buf in range(nbuf):
            one_block(bk * nbuf + buf, buf, bk > 0)
        return carry

    lax.fori_loop(0, nblocks // nbuf, block, 0)
    for rem in range(nblocks - (nblocks // nbuf) * nbuf):
        one_block(nblocks - nblocks % nbuf + rem, rem, True)
    for buf in range(nbuf):
        pltpu.make_async_copy(
            stages[buf], out_hbm.at[pl.ds(0, rows_blk)], sems[buf]
        ).wait()


@functools.cache
def _sc_materialize():
    return pl.kernel(
        _sc_materialize_body,
        out_type=jax.ShapeDtypeStruct((_NUM_HEADS * _SEQ, _SEQ), jnp.float32),
        mesh=plsc.VectorSubcoreMesh(core_axis_name="c", subcore_axis_name="s"),
        compiler_params=pltpu.CompilerParams(needs_layout_passes=False),
        scratch_types=[
            pltpu.VMEM((_BTAB,), jnp.int32),
            pltpu.VMEM((_NUM_BUCKETS, _NUM_HEADS), jnp.float32),
            pltpu.VMEM((_VROW,), jnp.float32),
            pltpu.VMEM((8, _SEQ), jnp.float32),
            pltpu.VMEM((8, _SEQ), jnp.float32),
            pltpu.VMEM((8, _SEQ), jnp.float32),
            pltpu.VMEM((8, _SEQ), jnp.float32),
            pltpu.VMEM((8, _SEQ), jnp.float32),
            pltpu.VMEM((8, _SEQ), jnp.float32),
            pltpu.SemaphoreType.DMA,
            pltpu.SemaphoreType.DMA,
            pltpu.SemaphoreType.DMA,
            pltpu.SemaphoreType.DMA,
            pltpu.SemaphoreType.DMA,
            pltpu.SemaphoreType.DMA,
        ],
    )


def kernel(lq, lk, embedding):
    delta = jnp.reshape(jnp.asarray(lk - lq, dtype=jnp.int32), (1,))
    btab = _bucket_tc(delta)
    out = _sc_materialize()(btab, embedding.astype(jnp.float32))
    return out.reshape(1, _NUM_HEADS, _SEQ, _SEQ)


# speedup vs baseline: 1.0386x; 1.0386x over previous
import functools
import math

import jax
import jax.numpy as jnp
from jax import lax
from jax.experimental import pallas as pl
from jax.experimental.pallas import tpu as pltpu
from jax.experimental.pallas import tpu_sc as plsc

_NUM_BUCKETS = 32
_NUM_HEADS = 16
_MAX_DIST = 128
_SEQ = 2048
_NDIAG = 2 * _SEQ - 1
_BROWS = 33
_BTAB = _BROWS * 128
_VROW = 4112
_HALF_ROWS = _SEQ // 2


def _bucket_tc_body(delta_ref, out_ref):
    flat = lax.broadcasted_iota(jnp.int32, (_BTAB,), 0)
    rel = flat - (_SEQ - 1) + delta_ref[0]
    half = _NUM_BUCKETS // 2
    buckets = (rel > 0).astype(jnp.int32) * half
    arel = jnp.abs(rel)
    max_exact = half // 2
    large = max_exact + (
        jnp.log(arel.astype(jnp.float32) / max_exact)
        / math.log(_MAX_DIST / max_exact)
        * (half - max_exact)
    ).astype(jnp.int32)
    large = jnp.minimum(large, half - 1)
    out_ref[...] = buckets + jnp.where(arel < max_exact, arel, large)


_bucket_tc = pl.pallas_call(
    _bucket_tc_body,
    out_shape=jax.ShapeDtypeStruct((_BTAB,), jnp.int32),
    in_specs=[pl.BlockSpec(memory_space=pltpu.SMEM)],
    out_specs=pl.BlockSpec(memory_space=pltpu.VMEM),
)


def _sc_materialize_body(
    btab_hbm, emb_hbm, out_hbm, btab_v, emb_v, vtab0,
    stage0, stage1, stage2, sem0, sem1, sem2
):
    stages = (stage0, stage1, stage2)
    sems = (sem0, sem1, sem2)
    cid = lax.axis_index("c")
    sid = lax.axis_index("s")
    wid = sid * 2 + cid
    head = wid // 2
    half = wid - head * 2
    c0 = pltpu.async_copy(btab_hbm, btab_v, sem0)
    c1 = pltpu.async_copy(emb_hbm, emb_v, sem1)
    c0.wait()
    c1.wait()
    iota = lax.iota(jnp.int32, 16)
    hvec = jnp.zeros((16,), jnp.int32) + head
    row0 = half * _HALF_ROWS

    @plsc.parallel_loop(0, _VROW // 16, unroll=4)
    def _build(k):
        bidx = btab_v[pl.ds(k * 16, 16)]
        vtab0[pl.ds(k * 16, 16)] = plsc.load_gather(emb_v, [bidx, hvec])

    nbuf = len(stages)
    rows_blk = 16

    def one_block(k, buf, wait):
        i0 = row0 + k * rows_blk
        stage = stages[buf]

        @pl.when(wait)
        def _wait_prev():
            pltpu.make_async_copy(
                stage, out_hbm.at[pl.ds(0, rows_blk)], sems[buf]
            ).wait()

        start0 = _SEQ - 1 - i0

        @plsc.parallel_loop(0, _SEQ // 16, unroll=8)
        def _fill(c):
            c16 = c * 16
            idx0 = start0 + c16 + iota
            for r in range(rows_blk):
                stage[r, pl.ds(c16, 16)] = plsc.load_gather(vtab0, [idx0 - r])

        pltpu.async_copy(
            stage, out_hbm.at[pl.ds(head * _SEQ + i0, rows_blk)], sems[buf]
        )

    nblocks = _HALF_ROWS // rows_blk

    def block(bk, carry):
        for buf in range(nbuf):
            one_block(bk * nbuf + buf, buf, bk > 0)
        return carry

    lax.fori_loop(0, nblocks // nbuf, block, 0)
    for rem in range(nblocks - (nblocks // nbuf) * nbuf):
        one_block(nblocks - nblocks % nbuf + rem, rem, True)
    for buf in range(nbuf):
        pltpu.make_async_copy(
            stages[buf], out_hbm.at[pl.ds(0, rows_blk)], sems[buf]
        ).wait()


@functools.cache
def _sc_materialize():
    return pl.kernel(
        _sc_materialize_body,
        out_type=jax.ShapeDtypeStruct((_NUM_HEADS * _SEQ, _SEQ), jnp.float32),
        mesh=plsc.VectorSubcoreMesh(core_axis_name="c", subcore_axis_name="s"),
        compiler_params=pltpu.CompilerParams(needs_layout_passes=False),
        scratch_types=[
            pltpu.VMEM((_BTAB,), jnp.int32),
            pltpu.VMEM((_NUM_BUCKETS, _NUM_HEADS), jnp.float32),
            pltpu.VMEM((_VROW,), jnp.float32),
            pltpu.VMEM((16, _SEQ), jnp.float32),
            pltpu.VMEM((16, _SEQ), jnp.float32),
            pltpu.VMEM((16, _SEQ), jnp.float32),
            pltpu.SemaphoreType.DMA,
            pltpu.SemaphoreType.DMA,
            pltpu.SemaphoreType.DMA,
        ],
    )


def kernel(lq, lk, embedding):
    delta = jnp.reshape(jnp.asarray(lk - lq, dtype=jnp.int32), (1,))
    btab = _bucket_tc(delta)
    out = _sc_materialize()(btab, embedding.astype(jnp.float32))
    return out.reshape(1, _NUM_HEADS, _SEQ, _SEQ)
